# Initial kernel scaffold; baseline (speedup 1.0000x reference)
import jax
import jax.numpy as jnp
from jax.experimental import pallas as pl


def kernel(hidden_states, table, w_v, w_k, g_k, g_h, s_cv, conv_w, hash_indices, offsets):
    raise NotImplementedError("write your pallas kernel here")


# speedup vs baseline: 1.5054x; 1.5054x over previous
import functools

import jax
import jax.numpy as jnp
from jax import lax
from jax.experimental import pallas as pl
from jax.experimental.pallas import tpu as pltpu
from jax.experimental.pallas import tpu_sc as plsc

B, S, H, DPH = 2, 4096, 8, 64
E_HID = H * DPH
HID = 1024
HC = 4
KS = 4

NC, NS = 2, 16
NW = NC * NS
TOTAL_IDX = B * S * H
PER_W = TOTAL_IDX // NW
CH = 128
N_CH = PER_W // CH
CPR = 4
N_ROUND = N_CH // CPR
ROWS_R = CPR * CH


def _sc_gather(idx3, table):
    mesh = plsc.VectorSubcoreMesh(core_axis_name="c", subcore_axis_name="s")

    @functools.partial(
        pl.kernel,
        mesh=mesh,
        compiler_params=pltpu.CompilerParams(use_tc_tiling_on_sc=False),
        out_type=jax.ShapeDtypeStruct((TOTAL_IDX, DPH), jnp.float32),
        scratch_types=[
            pltpu.VMEM((N_CH, CH), jnp.int32),
            pltpu.VMEM((ROWS_R, DPH), jnp.float32),
            pltpu.VMEM((ROWS_R, DPH), jnp.float32),
            pltpu.SemaphoreType.DMA,
            pltpu.SemaphoreType.DMA,
            pltpu.SemaphoreType.DMA,
            pltpu.SemaphoreType.DMA,
        ],
    )
    def gather_k(idx_hbm, table_hbm, out_hbm, idx_v, rows0, rows1,
                 gsem0, gsem1, osem0, osem1):
        wid = lax.axis_index("s") * NC + lax.axis_index("c")
        base = wid * PER_W
        pltpu.sync_copy(idx_hbm.at[wid], idx_v)
        bufs = (rows0, rows1)
        gsems = (gsem0, gsem1)
        osems = (osem0, osem1)
        out_cps = []
        for r in range(N_ROUND):
            b = r % 2
            if r >= 2:
                out_cps[r - 2].wait()
            gcps = []
            for j in range(CPR):
                ch = r * CPR + j
                gcps.append(pltpu.async_copy(
                    table_hbm.at[idx_v.at[ch]],
                    bufs[b].at[pl.ds(j * CH, CH)],
                    gsems[b]))
            for c in gcps:
                c.wait()
            out_cps.append(pltpu.async_copy(
                bufs[b], out_hbm.at[pl.ds(base + r * ROWS_R, ROWS_R)],
                osems[b]))
        out_cps[-2].wait()
        out_cps[-1].wait()

    return gather_k(idx3, table)


BS = 512
NBLK = S // BS


def _tc_body(emb_ref, hid_ref, wv_ref, wk_ref, gk_ref, gh_ref, scv_ref,
             cw_ref, out_ref, carry_ref):
    i = pl.program_id(1)

    @pl.when(i == 0)
    def _():
        carry_ref[...] = jnp.zeros_like(carry_ref)

    e = emb_ref[0].astype(jnp.bfloat16)
    h = hid_ref[0]
    dn = (((1,), (1,)), ((), ()))
    value = lax.dot_general(e, wv_ref[...], dn,
                            preferred_element_type=jnp.float32)
    for m in range(HC):
        key = lax.dot_general(e, wk_ref[m], dn,
                              preferred_element_type=jnp.float32)
        nk = key * lax.rsqrt(jnp.mean(key * key, axis=-1, keepdims=True)
                             + 1e-6) * gk_ref[m][None, :]
        hm = h[:, m * HID:(m + 1) * HID]
        nq = hm * lax.rsqrt(jnp.mean(hm * hm, axis=-1, keepdims=True)
                            + 1e-6) * gh_ref[m][None, :]
        score = jnp.sum(nk * nq, axis=-1, keepdims=True) * (1.0 / 32.0)
        gate = jnp.sqrt(jnp.clip(jnp.abs(score), 1e-6, None)) * jnp.sign(score)
        gate = jax.nn.sigmoid(gate)
        gated = gate * value
        xm = gated * lax.rsqrt(jnp.mean(gated * gated, axis=-1, keepdims=True)
                               + 1e-5) * scv_ref[m][None, :]
        prev = carry_ref[m, 5:8, :]
        xfull = jnp.concatenate([prev, xm], axis=0)
        conv = (cw_ref[m, 0][None, :] * xfull[0:BS] +
                cw_ref[m, 1][None, :] * xfull[1:BS + 1] +
                cw_ref[m, 2][None, :] * xfull[2:BS + 2] +
                cw_ref[m, 3][None, :] * xfull[3:BS + 3])
        y = conv * jax.nn.sigmoid(conv) + gated
        out_ref[0, :, m * HID:(m + 1) * HID] = hm + y
        carry_ref[m] = xm[BS - 8:BS, :]


def _tc_dense(emb, hid2, wv_b, wk_b, g_k, g_h, s_cv, cwt):
    return pl.pallas_call(
        _tc_body,
        grid=(B, NBLK),
        in_specs=[
            pl.BlockSpec((1, BS, E_HID), lambda b, i: (b, i, 0)),
            pl.BlockSpec((1, BS, HC * HID), lambda b, i: (b, i, 0)),
            pl.BlockSpec((HID, E_HID), lambda b, i: (0, 0)),
            pl.BlockSpec((HC, HID, E_HID), lambda b, i: (0, 0, 0)),
            pl.BlockSpec((HC, HID), lambda b, i: (0, 0)),
            pl.BlockSpec((HC, HID), lambda b, i: (0, 0)),
            pl.BlockSpec((HC, HID), lambda b, i: (0, 0)),
            pl.BlockSpec((HC, KS, HID), lambda b, i: (0, 0, 0)),
        ],
        out_specs=pl.BlockSpec((1, BS, HC * HID), lambda b, i: (b, i, 0)),
        out_shape=jax.ShapeDtypeStruct((B, S, HC * HID), jnp.float32),
        scratch_shapes=[pltpu.VMEM((HC, 8, HID), jnp.float32)],
    )(emb, hid2, wv_b, wk_b, g_k, g_h, s_cv, cwt)


def kernel(hidden_states, table, w_v, w_k, g_k, g_h, s_cv, conv_w,
           hash_indices, offsets):
    idx3 = (hash_indices + offsets[None, None, :]).reshape(NW, N_CH, CH)
    rows = _sc_gather(idx3, table)
    emb = rows.reshape(B, S, E_HID)
    hid2 = hidden_states.reshape(B, S, HC * HID)
    cwt = jnp.transpose(conv_w.reshape(HC, HID, KS), (0, 2, 1))
    out = _tc_dense(emb, hid2, w_v.astype(jnp.bfloat16),
                    w_k.astype(jnp.bfloat16), g_k, g_h, s_cv, cwt)
    return out.reshape(B, S, HC, HID)


# speedup vs baseline: 1.5968x; 1.0607x over previous
import functools

import jax
import jax.numpy as jnp
from jax import lax
from jax.experimental import pallas as pl
from jax.experimental.pallas import tpu as pltpu
from jax.experimental.pallas import tpu_sc as plsc

B, S, H, DPH = 2, 4096, 8, 64
E_HID = H * DPH
HID = 1024
HC = 4
KS = 4
V_ROWS = 2000126

BLKV = 16384
NBLKV = -(-V_ROWS // BLKV)
V_PAD = NBLKV * BLKV


def _transpose_body(tt_ref, out_ref):
    out_ref[...] = jnp.swapaxes(tt_ref[...], 0, 1)


def _tc_transpose(table_t):
    return pl.pallas_call(
        _transpose_body,
        grid=(NBLKV,),
        in_specs=[pl.BlockSpec((DPH, BLKV), lambda i: (0, i))],
        out_specs=pl.BlockSpec((BLKV, DPH), lambda i: (i, 0)),
        out_shape=jax.ShapeDtypeStruct((V_PAD, DPH), jnp.float32),
    )(table_t)


NC, NS = 2, 16
NW = NC * NS
TOTAL_IDX = B * S * H
PER_W = TOTAL_IDX // NW
CH = 128
N_CH = PER_W // CH
CPR = 4
N_ROUND = N_CH // CPR
ROWS_R = CPR * CH


def _sc_gather(idx3, table_rm):
    mesh = plsc.VectorSubcoreMesh(core_axis_name="c", subcore_axis_name="s")

    @functools.partial(
        pl.kernel,
        mesh=mesh,
        compiler_params=pltpu.CompilerParams(use_tc_tiling_on_sc=False),
        out_type=jax.ShapeDtypeStruct((TOTAL_IDX, DPH), jnp.float32),
        scratch_types=[
            pltpu.VMEM((N_CH, CH), jnp.int32),
            pltpu.VMEM((ROWS_R, DPH), jnp.float32),
            pltpu.VMEM((ROWS_R, DPH), jnp.float32),
            pltpu.SemaphoreType.DMA,
            pltpu.SemaphoreType.DMA,
            pltpu.SemaphoreType.DMA,
            pltpu.SemaphoreType.DMA,
        ],
    )
    def gather_k(idx_hbm, table_hbm, out_hbm, idx_v, rows0, rows1,
                 gsem0, gsem1, osem0, osem1):
        wid = lax.axis_index("s") * NC + lax.axis_index("c")
        base = wid * PER_W
        pltpu.sync_copy(idx_hbm.at[wid], idx_v)
        bufs = (rows0, rows1)
        gsems = (gsem0, gsem1)
        osems = (osem0, osem1)
        out_cps = []
        for r in range(N_ROUND):
            b = r % 2
            if r >= 2:
                out_cps[r - 2].wait()
            gcps = []
            for j in range(CPR):
                ch = r * CPR + j
                gcps.append(pltpu.async_copy(
                    table_hbm.at[idx_v.at[ch]],
                    bufs[b].at[pl.ds(j * CH, CH)],
                    gsems[b]))
            for c in gcps:
                c.wait()
            out_cps.append(pltpu.async_copy(
                bufs[b], out_hbm.at[pl.ds(base + r * ROWS_R, ROWS_R)],
                osems[b]))
        out_cps[-2].wait()
        out_cps[-1].wait()

    return gather_k(idx3, table_rm)


BS = 512
NBLK = S // BS


def _tc_body(emb_ref, hid_ref, wv_ref, wk_ref, gk_ref, gh_ref, scv_ref,
             cw_ref, out_ref, carry_ref):
    i = pl.program_id(1)

    @pl.when(i == 0)
    def _():
        carry_ref[...] = jnp.zeros_like(carry_ref)

    e = emb_ref[0].astype(jnp.bfloat16)
    dn = (((1,), (1,)), ((), ()))
    value = lax.dot_general(e, wv_ref[...], dn,
                            preferred_element_type=jnp.float32)
    for m in range(HC):
        key = lax.dot_general(e, wk_ref[m], dn,
                              preferred_element_type=jnp.float32)
        nk = key * lax.rsqrt(jnp.mean(key * key, axis=-1, keepdims=True)
                             + 1e-6) * gk_ref[m][None, :]
        hm = hid_ref[0, :, m, :]
        nq = hm * lax.rsqrt(jnp.mean(hm * hm, axis=-1, keepdims=True)
                            + 1e-6) * gh_ref[m][None, :]
        score = jnp.sum(nk * nq, axis=-1, keepdims=True) * (1.0 / 32.0)
        gate = jnp.sqrt(jnp.clip(jnp.abs(score), 1e-6, None)) * jnp.sign(score)
        gate = jax.nn.sigmoid(gate)
        gated = gate * value
        xm = gated * lax.rsqrt(jnp.mean(gated * gated, axis=-1, keepdims=True)
                               + 1e-5) * scv_ref[m][None, :]
        prev = carry_ref[m, 5:8, :]
        xfull = jnp.concatenate([prev, xm], axis=0)
        conv = (cw_ref[0, m][None, :] * xfull[0:BS] +
                cw_ref[1, m][None, :] * xfull[1:BS + 1] +
                cw_ref[2, m][None, :] * xfull[2:BS + 2] +
                cw_ref[3, m][None, :] * xfull[3:BS + 3])
        y = conv * jax.nn.sigmoid(conv) + gated
        out_ref[0, :, m, :] = hm + y
        carry_ref[m] = xm[BS - 8:BS, :]


def _tc_dense(emb, hidden, wv_b, wk_b, g_k, g_h, s_cv, cwt):
    return pl.pallas_call(
        _tc_body,
        grid=(B, NBLK),
        in_specs=[
            pl.BlockSpec((1, BS, E_HID), lambda b, i: (b, i, 0)),
            pl.BlockSpec((1, BS, HC, HID), lambda b, i: (b, i, 0, 0)),
            pl.BlockSpec((HID, E_HID), lambda b, i: (0, 0)),
            pl.BlockSpec((HC, HID, E_HID), lambda b, i: (0, 0, 0)),
            pl.BlockSpec((HC, HID), lambda b, i: (0, 0)),
            pl.BlockSpec((HC, HID), lambda b, i: (0, 0)),
            pl.BlockSpec((HC, HID), lambda b, i: (0, 0)),
            pl.BlockSpec((KS, HC, HID), lambda b, i: (0, 0, 0)),
        ],
        out_specs=pl.BlockSpec((1, BS, HC, HID), lambda b, i: (b, i, 0, 0)),
        out_shape=jax.ShapeDtypeStruct((B, S, HC, HID), jnp.float32),
        scratch_shapes=[pltpu.VMEM((HC, 8, HID), jnp.float32)],
    )(emb, hidden, wv_b, wk_b, g_k, g_h, s_cv, cwt)


def kernel(hidden_states, table, w_v, w_k, g_k, g_h, s_cv, conv_w,
           hash_indices, offsets):
    table_rm = _tc_transpose(table.T)
    idx3 = (hash_indices + offsets[None, None, :]).reshape(NW, N_CH, CH)
    rows = _sc_gather(idx3, table_rm)
    emb = rows.reshape(B, S, E_HID)
    cwt = conv_w.T.reshape(KS, HC, HID)
    return _tc_dense(emb, hidden_states, w_v.astype(jnp.bfloat16),
                     w_k.astype(jnp.bfloat16), g_k, g_h, s_cv, cwt)


# speedup vs baseline: 2.9650x; 1.8569x over previous
import functools

import jax
import jax.numpy as jnp
from jax import lax
from jax.experimental import pallas as pl
from jax.experimental.pallas import tpu as pltpu
from jax.experimental.pallas import tpu_sc as plsc

B, S, H, DPH = 2, 4096, 8, 64
E_HID = H * DPH
HID = 1024
HC = 4
KS = 4
V_ROWS = 2000126

BLKV = 16384
NBLKV = -(-V_ROWS // BLKV)
V_PAD = NBLKV * BLKV


def _transpose_body(tt_ref, out_ref):
    t = jnp.swapaxes(tt_ref[...], 0, 1)
    out_ref[...] = jnp.concatenate([t, jnp.zeros_like(t)], axis=1)


def _tc_transpose(table_t):
    return pl.pallas_call(
        _transpose_body,
        grid=(NBLKV,),
        in_specs=[pl.BlockSpec((DPH, BLKV), lambda i: (0, i))],
        out_specs=pl.BlockSpec((BLKV, 2 * DPH), lambda i: (i, 0)),
        out_shape=jax.ShapeDtypeStruct((V_PAD, 2 * DPH), jnp.float32),
    )(table_t)


NC, NS = 2, 16
NW = NC * NS
TOTAL_IDX = B * S * H
PER_W = TOTAL_IDX // NW
CH = 128
N_CH = PER_W // CH
CPR = 2
N_ROUND = N_CH // CPR
ROWS_R = CPR * CH


def _sc_gather(idx3, table_rm):
    mesh = plsc.VectorSubcoreMesh(core_axis_name="c", subcore_axis_name="s")

    @functools.partial(
        pl.kernel,
        mesh=mesh,
        compiler_params=pltpu.CompilerParams(use_tc_tiling_on_sc=False),
        out_type=jax.ShapeDtypeStruct((TOTAL_IDX, DPH), jnp.float32),
        scratch_types=[
            pltpu.VMEM((N_CH, CH), jnp.int32),
            pltpu.VMEM((ROWS_R, 2 * DPH), jnp.float32),
            pltpu.VMEM((ROWS_R, 2 * DPH), jnp.float32),
            pltpu.SemaphoreType.DMA,
            pltpu.SemaphoreType.DMA,
            pltpu.SemaphoreType.DMA,
            pltpu.SemaphoreType.DMA,
        ],
    )
    def gather_k(idx_hbm, table_hbm, out_hbm, idx_v, rows0, rows1,
                 gsem0, gsem1, osem0, osem1):
        wid = lax.axis_index("s") * NC + lax.axis_index("c")
        base = wid * PER_W
        pltpu.sync_copy(idx_hbm.at[wid], idx_v)
        bufs = (rows0, rows1)
        gsems = (gsem0, gsem1)
        osems = (osem0, osem1)
        out_cps = []
        for r in range(N_ROUND):
            b = r % 2
            if r >= 2:
                out_cps[r - 2].wait()
            gcps = []
            for j in range(CPR):
                ch = r * CPR + j
                gcps.append(pltpu.async_copy(
                    table_hbm.at[idx_v.at[ch]],
                    bufs[b].at[pl.ds(j * CH, CH)],
                    gsems[b]))
            for c in gcps:
                c.wait()
            out_cps.append(pltpu.async_copy(
                bufs[b].at[:, pl.ds(0, DPH)],
                out_hbm.at[pl.ds(base + r * ROWS_R, ROWS_R)],
                osems[b]))
        out_cps[-2].wait()
        out_cps[-1].wait()

    return gather_k(idx3, table_rm)


BS = 512
NBLK = S // BS


def _tc_body(emb_ref, hid_ref, wv_ref, wk_ref, gk_ref, gh_ref, scv_ref,
             cw_ref, out_ref, carry_ref):
    i = pl.program_id(1)

    @pl.when(i == 0)
    def _():
        carry_ref[...] = jnp.zeros_like(carry_ref)

    e = emb_ref[0].astype(jnp.bfloat16)
    dn = (((1,), (1,)), ((), ()))
    value = lax.dot_general(e, wv_ref[...], dn,
                            preferred_element_type=jnp.float32)
    for m in range(HC):
        key = lax.dot_general(e, wk_ref[m], dn,
                              preferred_element_type=jnp.float32)
        nk = key * lax.rsqrt(jnp.mean(key * key, axis=-1, keepdims=True)
                             + 1e-6) * gk_ref[m][None, :]
        hm = hid_ref[0, :, m, :]
        nq = hm * lax.rsqrt(jnp.mean(hm * hm, axis=-1, keepdims=True)
                            + 1e-6) * gh_ref[m][None, :]
        score = jnp.sum(nk * nq, axis=-1, keepdims=True) * (1.0 / 32.0)
        gate = jnp.sqrt(jnp.clip(jnp.abs(score), 1e-6, None)) * jnp.sign(score)
        gate = jax.nn.sigmoid(gate)
        gated = gate * value
        xm = gated * lax.rsqrt(jnp.mean(gated * gated, axis=-1, keepdims=True)
                               + 1e-5) * scv_ref[m][None, :]
        prev = carry_ref[m, 5:8, :]
        xfull = jnp.concatenate([prev, xm], axis=0)
        conv = (cw_ref[0, m][None, :] * xfull[0:BS] +
                cw_ref[1, m][None, :] * xfull[1:BS + 1] +
                cw_ref[2, m][None, :] * xfull[2:BS + 2] +
                cw_ref[3, m][None, :] * xfull[3:BS + 3])
        y = conv * jax.nn.sigmoid(conv) + gated
        out_ref[0, :, m, :] = hm + y
        carry_ref[m] = xm[BS - 8:BS, :]


def _tc_dense(emb, hidden, wv_b, wk_b, g_k, g_h, s_cv, cwt):
    return pl.pallas_call(
        _tc_body,
        grid=(B, NBLK),
        in_specs=[
            pl.BlockSpec((1, BS, E_HID), lambda b, i: (b, i, 0)),
            pl.BlockSpec((1, BS, HC, HID), lambda b, i: (b, i, 0, 0)),
            pl.BlockSpec((HID, E_HID), lambda b, i: (0, 0)),
            pl.BlockSpec((HC, HID, E_HID), lambda b, i: (0, 0, 0)),
            pl.BlockSpec((HC, HID), lambda b, i: (0, 0)),
            pl.BlockSpec((HC, HID), lambda b, i: (0, 0)),
            pl.BlockSpec((HC, HID), lambda b, i: (0, 0)),
            pl.BlockSpec((KS, HC, HID), lambda b, i: (0, 0, 0)),
        ],
        out_specs=pl.BlockSpec((1, BS, HC, HID), lambda b, i: (b, i, 0, 0)),
        out_shape=jax.ShapeDtypeStruct((B, S, HC, HID), jnp.float32),
        scratch_shapes=[pltpu.VMEM((HC, 8, HID), jnp.float32)],
    )(emb, hidden, wv_b, wk_b, g_k, g_h, s_cv, cwt)


def kernel(hidden_states, table, w_v, w_k, g_k, g_h, s_cv, conv_w,
           hash_indices, offsets):
    table_rm = _tc_transpose(table.T)
    idx3 = (hash_indices + offsets[None, None, :]).reshape(NW, N_CH, CH)
    rows = _sc_gather(idx3, table_rm)
    emb = rows.reshape(B, S, E_HID)
    cwt = conv_w.T.reshape(KS, HC, HID)
    return _tc_dense(emb, hidden_states, w_v.astype(jnp.bfloat16),
                     w_k.astype(jnp.bfloat16), g_k, g_h, s_cv, cwt)


# speedup vs baseline: 3.0961x; 1.0442x over previous
import functools

import jax
import jax.numpy as jnp
from jax import lax
from jax.experimental import pallas as pl
from jax.experimental.pallas import tpu as pltpu
from jax.experimental.pallas import tpu_sc as plsc

B, S, H, DPH = 2, 4096, 8, 64
E_HID = H * DPH
HID = 1024
HC = 4
KS = 4
V_ROWS = 2000126

BLKV = 16384
PAIR_BLK = 61
PAIR_OFF = PAIR_BLK * BLKV
NBLKV = 62
N_LINES = NBLKV * BLKV


def _transpose_body(ttl_ref, ttr_ref, out_ref):
    tl = jnp.swapaxes(ttl_ref[...], 0, 1)
    tr = jnp.swapaxes(ttr_ref[...], 0, 1)
    out_ref[...] = jnp.concatenate([tl, tr], axis=1)


def _tc_transpose(table_t):
    return pl.pallas_call(
        _transpose_body,
        grid=(NBLKV,),
        in_specs=[
            pl.BlockSpec((DPH, BLKV), lambda i: (0, i)),
            pl.BlockSpec((DPH, BLKV), lambda i: (0, i + PAIR_BLK)),
        ],
        out_specs=pl.BlockSpec((BLKV, 2 * DPH), lambda i: (i, 0)),
        out_shape=jax.ShapeDtypeStruct((N_LINES, 2 * DPH), jnp.float32),
    )(table_t, table_t)


NC, NS = 2, 16
NW = NC * NS
TOTAL_IDX = B * S * H
PER_W = TOTAL_IDX // NW
CH = 128
N_CH = PER_W // CH
CPR = 2
N_ROUND = N_CH // CPR
ROWS_R = CPR * CH


def _sc_gather(idx3, table_rm):
    mesh = plsc.VectorSubcoreMesh(core_axis_name="c", subcore_axis_name="s")

    @functools.partial(
        pl.kernel,
        mesh=mesh,
        compiler_params=pltpu.CompilerParams(use_tc_tiling_on_sc=False),
        out_type=jax.ShapeDtypeStruct((TOTAL_IDX, DPH), jnp.float32),
        scratch_types=[
            pltpu.VMEM((N_CH, CH), jnp.int32),
            pltpu.VMEM((ROWS_R, 2 * DPH), jnp.float32),
            pltpu.VMEM((ROWS_R, 2 * DPH), jnp.float32),
            pltpu.SemaphoreType.DMA,
            pltpu.SemaphoreType.DMA,
            pltpu.SemaphoreType.DMA,
            pltpu.SemaphoreType.DMA,
        ],
    )
    def gather_k(idx_hbm, table_hbm, out_hbm, idx_v, rows0, rows1,
                 gsem0, gsem1, osem0, osem1):
        wid = lax.axis_index("s") * NC + lax.axis_index("c")
        base = wid * PER_W
        half = jnp.where(wid >= NW // 2, DPH, 0)
        pltpu.sync_copy(idx_hbm.at[wid], idx_v)
        bufs = (rows0, rows1)
        gsems = (gsem0, gsem1)
        osems = (osem0, osem1)
        out_cps = []
        for r in range(N_ROUND):
            b = r % 2
            if r >= 2:
                out_cps[r - 2].wait()
            gcps = []
            for j in range(CPR):
                ch = r * CPR + j
                gcps.append(pltpu.async_copy(
                    table_hbm.at[idx_v.at[ch]],
                    bufs[b].at[pl.ds(j * CH, CH)],
                    gsems[b]))
            for c in gcps:
                c.wait()
            out_cps.append(pltpu.async_copy(
                bufs[b].at[:, pl.ds(half, DPH)],
                out_hbm.at[pl.ds(base + r * ROWS_R, ROWS_R)],
                osems[b]))
        out_cps[-2].wait()
        out_cps[-1].wait()

    return gather_k(idx3, table_rm)


BS = 512
NBLK = S // BS


def _tc_body(emb_ref, hid_ref, wv_ref, wk_ref, gk_ref, gh_ref, scv_ref,
             cw_ref, out_ref, carry_ref):
    i = pl.program_id(1)

    @pl.when(i == 0)
    def _():
        carry_ref[...] = jnp.zeros_like(carry_ref)

    e = emb_ref[0].astype(jnp.bfloat16)
    dn = (((1,), (1,)), ((), ()))
    value = lax.dot_general(e, wv_ref[...], dn,
                            preferred_element_type=jnp.float32)
    for m in range(HC):
        key = lax.dot_general(e, wk_ref[m], dn,
                              preferred_element_type=jnp.float32)
        nk = key * lax.rsqrt(jnp.mean(key * key, axis=-1, keepdims=True)
                             + 1e-6) * gk_ref[m][None, :]
        hm = hid_ref[0, :, m, :]
        nq = hm * lax.rsqrt(jnp.mean(hm * hm, axis=-1, keepdims=True)
                            + 1e-6) * gh_ref[m][None, :]
        score = jnp.sum(nk * nq, axis=-1, keepdims=True) * (1.0 / 32.0)
        gate = jnp.sqrt(jnp.clip(jnp.abs(score), 1e-6, None)) * jnp.sign(score)
        gate = jax.nn.sigmoid(gate)
        gated = gate * value
        xm = gated * lax.rsqrt(jnp.mean(gated * gated, axis=-1, keepdims=True)
                               + 1e-5) * scv_ref[m][None, :]
        prev = carry_ref[m, 5:8, :]
        xfull = jnp.concatenate([prev, xm], axis=0)
        conv = (cw_ref[0, m][None, :] * xfull[0:BS] +
                cw_ref[1, m][None, :] * xfull[1:BS + 1] +
                cw_ref[2, m][None, :] * xfull[2:BS + 2] +
                cw_ref[3, m][None, :] * xfull[3:BS + 3])
        y = conv * jax.nn.sigmoid(conv) + gated
        out_ref[0, :, m, :] = hm + y
        carry_ref[m] = xm[BS - 8:BS, :]


def _tc_dense(emb, hidden, wv_b, wk_b, g_k, g_h, s_cv, cwt):
    return pl.pallas_call(
        _tc_body,
        grid=(B, NBLK),
        in_specs=[
            pl.BlockSpec((1, BS, E_HID), lambda b, i: (b, i, 0)),
            pl.BlockSpec((1, BS, HC, HID), lambda b, i: (b, i, 0, 0)),
            pl.BlockSpec((HID, E_HID), lambda b, i: (0, 0)),
            pl.BlockSpec((HC, HID, E_HID), lambda b, i: (0, 0, 0)),
            pl.BlockSpec((HC, HID), lambda b, i: (0, 0)),
            pl.BlockSpec((HC, HID), lambda b, i: (0, 0)),
            pl.BlockSpec((HC, HID), lambda b, i: (0, 0)),
            pl.BlockSpec((KS, HC, HID), lambda b, i: (0, 0, 0)),
        ],
        out_specs=pl.BlockSpec((1, BS, HC, HID), lambda b, i: (b, i, 0, 0)),
        out_shape=jax.ShapeDtypeStruct((B, S, HC, HID), jnp.float32),
        scratch_shapes=[pltpu.VMEM((HC, 8, HID), jnp.float32)],
    )(emb, hidden, wv_b, wk_b, g_k, g_h, s_cv, cwt)


def kernel(hidden_states, table, w_v, w_k, g_k, g_h, s_cv, conv_w,
           hash_indices, offsets):
    table_rm = _tc_transpose(table.T)
    off_adj = offsets - jnp.where(jnp.arange(H) >= H // 2, PAIR_OFF, 0)
    sh = hash_indices + off_adj[None, None, :]
    idx3 = jnp.transpose(sh, (2, 0, 1)).reshape(NW, N_CH, CH)
    rows = _sc_gather(idx3, table_rm)
    emb = jnp.transpose(rows.reshape(H, B, S, DPH),
                        (1, 2, 0, 3)).reshape(B, S, E_HID)
    cwt = conv_w.T.reshape(KS, HC, HID)
    return _tc_dense(emb, hidden_states, w_v.astype(jnp.bfloat16),
                     w_k.astype(jnp.bfloat16), g_k, g_h, s_cv, cwt)


# speedup vs baseline: 3.1386x; 1.0137x over previous
import functools

import jax
import jax.numpy as jnp
from jax import lax
from jax.experimental import pallas as pl
from jax.experimental.pallas import tpu as pltpu
from jax.experimental.pallas import tpu_sc as plsc

B, S, H, DPH = 2, 4096, 8, 64
E_HID = H * DPH
HID = 1024
HC = 4
KS = 4
V_ROWS = 2000126

BLKV = 16384
PAIR_BLK = 61
PAIR_OFF = PAIR_BLK * BLKV
NBLKV = 62
N_LINES = NBLKV * BLKV


def _transpose_body(ttl_ref, ttr_ref, out_ref):
    out_ref[:, 0:DPH] = jnp.swapaxes(ttl_ref[...], 0, 1)
    out_ref[:, DPH:2 * DPH] = jnp.swapaxes(ttr_ref[...], 0, 1)


def _tc_transpose(table_t):
    return pl.pallas_call(
        _transpose_body,
        grid=(NBLKV,),
        in_specs=[
            pl.BlockSpec((DPH, BLKV), lambda i: (0, i)),
            pl.BlockSpec((DPH, BLKV), lambda i: (0, i + PAIR_BLK)),
        ],
        out_specs=pl.BlockSpec((BLKV, 2 * DPH), lambda i: (i, 0)),
        out_shape=jax.ShapeDtypeStruct((N_LINES, 2 * DPH), jnp.float32),
    )(table_t, table_t)


NC, NS = 2, 16
NW = NC * NS
TOTAL_IDX = B * S * H
PER_W = TOTAL_IDX // NW
CH = 128
N_CH = PER_W // CH
CPR = 2
N_ROUND = N_CH // CPR
ROWS_R = CPR * CH


def _sc_gather(idx3, table_rm):
    mesh = plsc.VectorSubcoreMesh(core_axis_name="c", subcore_axis_name="s")

    @functools.partial(
        pl.kernel,
        mesh=mesh,
        compiler_params=pltpu.CompilerParams(use_tc_tiling_on_sc=False),
        out_type=jax.ShapeDtypeStruct((TOTAL_IDX, DPH), jnp.float32),
        scratch_types=[
            pltpu.VMEM((N_CH, CH), jnp.int32),
            pltpu.VMEM((ROWS_R, 2 * DPH), jnp.float32),
            pltpu.VMEM((ROWS_R, 2 * DPH), jnp.float32),
            pltpu.SemaphoreType.DMA,
            pltpu.SemaphoreType.DMA,
            pltpu.SemaphoreType.DMA,
            pltpu.SemaphoreType.DMA,
        ],
    )
    def gather_k(idx_hbm, table_hbm, out_hbm, idx_v, rows0, rows1,
                 gsem0, gsem1, osem0, osem1):
        wid = lax.axis_index("s") * NC + lax.axis_index("c")
        base = wid * PER_W
        half = jnp.where(wid >= NW // 2, DPH, 0)
        pltpu.sync_copy(idx_hbm.at[wid], idx_v)
        bufs = (rows0, rows1)
        gsems = (gsem0, gsem1)
        osems = (osem0, osem1)
        out_cps = []
        for r in range(N_ROUND):
            b = r % 2
            if r >= 2:
                out_cps[r - 2].wait()
            gcps = []
            for j in range(CPR):
                ch = r * CPR + j
                gcps.append(pltpu.async_copy(
                    table_hbm.at[idx_v.at[ch]],
                    bufs[b].at[pl.ds(j * CH, CH)],
                    gsems[b]))
            for c in gcps:
                c.wait()
            out_cps.append(pltpu.async_copy(
                bufs[b].at[:, pl.ds(half, DPH)],
                out_hbm.at[pl.ds(base + r * ROWS_R, ROWS_R)],
                osems[b]))
        out_cps[-2].wait()
        out_cps[-1].wait()

    return gather_k(idx3, table_rm)


BS = 512
NBLK = S // BS


def _tc_body(emb_ref, hid_ref, wv_ref, wk_ref, gg_ref, scv_ref,
             cw_ref, out_ref, carry_ref):
    i = pl.program_id(1)

    @pl.when(i == 0)
    def _():
        carry_ref[...] = jnp.zeros_like(carry_ref)

    e = emb_ref[0].astype(jnp.bfloat16)
    dn = (((1,), (1,)), ((), ()))
    value = lax.dot_general(e, wv_ref[...], dn,
                            preferred_element_type=jnp.float32)
    mv2 = jnp.mean(value * value, axis=-1, keepdims=True)
    for m in range(HC):
        key = lax.dot_general(e, wk_ref[m], dn,
                              preferred_element_type=jnp.float32)
        rsk = lax.rsqrt(jnp.mean(key * key, axis=-1, keepdims=True) + 1e-6)
        hm = hid_ref[0, :, m, :]
        rsh = lax.rsqrt(jnp.mean(hm * hm, axis=-1, keepdims=True) + 1e-6)
        raw = jnp.sum(key * hm * gg_ref[m][None, :], axis=-1, keepdims=True)
        score = raw * rsk * rsh * (1.0 / 32.0)
        gate = jnp.sqrt(jnp.clip(jnp.abs(score), 1e-6, None)) * jnp.sign(score)
        gate = jax.nn.sigmoid(gate)
        gated = gate * value
        gr = gate * lax.rsqrt(gate * gate * mv2 + 1e-5)
        xm = (gr * value) * scv_ref[m][None, :]
        prev = carry_ref[m, 5:8, :]
        xfull = jnp.concatenate([prev, xm], axis=0)
        conv = (cw_ref[0, m][None, :] * xfull[0:BS] +
                cw_ref[1, m][None, :] * xfull[1:BS + 1] +
                cw_ref[2, m][None, :] * xfull[2:BS + 2] +
                cw_ref[3, m][None, :] * xfull[3:BS + 3])
        y = conv * jax.nn.sigmoid(conv) + gated
        out_ref[0, :, m, :] = hm + y
        carry_ref[m] = xm[BS - 8:BS, :]


def _tc_dense(emb, hidden, wv_b, wk_b, gg, s_cv, cwt):
    return pl.pallas_call(
        _tc_body,
        grid=(B, NBLK),
        in_specs=[
            pl.BlockSpec((1, BS, E_HID), lambda b, i: (b, i, 0)),
            pl.BlockSpec((1, BS, HC, HID), lambda b, i: (b, i, 0, 0)),
            pl.BlockSpec((HID, E_HID), lambda b, i: (0, 0)),
            pl.BlockSpec((HC, HID, E_HID), lambda b, i: (0, 0, 0)),
            pl.BlockSpec((HC, HID), lambda b, i: (0, 0)),
            pl.BlockSpec((HC, HID), lambda b, i: (0, 0)),
            pl.BlockSpec((KS, HC, HID), lambda b, i: (0, 0, 0)),
        ],
        out_specs=pl.BlockSpec((1, BS, HC, HID), lambda b, i: (b, i, 0, 0)),
        out_shape=jax.ShapeDtypeStruct((B, S, HC, HID), jnp.float32),
        scratch_shapes=[pltpu.VMEM((HC, 8, HID), jnp.float32)],
    )(emb, hidden, wv_b, wk_b, gg, s_cv, cwt)


def kernel(hidden_states, table, w_v, w_k, g_k, g_h, s_cv, conv_w,
           hash_indices, offsets):
    table_rm = _tc_transpose(table.T)
    off_adj = offsets - jnp.where(jnp.arange(H) >= H // 2, PAIR_OFF, 0)
    sh = hash_indices + off_adj[None, None, :]
    idx3 = jnp.transpose(sh, (2, 0, 1)).reshape(NW, N_CH, CH)
    rows = _sc_gather(idx3, table_rm)
    emb = jnp.transpose(rows.reshape(H, B, S, DPH),
                        (1, 2, 0, 3)).reshape(B, S, E_HID)
    cwt = conv_w.T.reshape(KS, HC, HID)
    return _tc_dense(emb, hidden_states, w_v.astype(jnp.bfloat16),
                     w_k.astype(jnp.bfloat16), g_k * g_h, s_cv, cwt)
